# baseline (speedup 1.0000x reference)
import jax
import jax.numpy as jnp
from jax import lax
from jax.experimental import pallas as pl
from jax.experimental.pallas import tpu as pltpu
from jax.experimental.pallas import tpu_sc as plsc

B = 4096
L = 200
VOCAB = 1000000
EMBED_DIM = 64
PAD_DIM = 128

_info = plsc.get_sparse_core_info()
_NC = _info.num_cores
_NS = _info.num_subcores
_NW = _NC * _NS

_TC = 256
_NFULL = VOCAB // _TC
_KPW = _NFULL // _NW
_NEXTRA = _NFULL - _KPW * _NW
_TAIL = VOCAB - _NFULL * _TC


def _transpose_kernel(wt_hbm, out_hbm, in_v0, in_v1, out_v0, out_v1, in_t, out_t,
                      sem_i0, sem_i1, sem_o0, sem_o1, sem_t):
    in_v = (in_v0, in_v1)
    out_v = (out_v0, out_v1)
    sem_i = (sem_i0, sem_i1)
    sem_o = (sem_o0, sem_o1)
    wid = lax.axis_index("s") * _NC + lax.axis_index("c")

    iota64 = lax.iota(jnp.int32, 16) * EMBED_DIM

    def in_start(start, b):
        pltpu.async_copy(wt_hbm.at[:, pl.ds(start, _TC)], in_v[b], sem_i[b])

    def in_wait(start, b):
        pltpu.make_async_copy(
            wt_hbm.at[:, pl.ds(start, _TC)], in_v[b], sem_i[b]
        ).wait()

    def out_start(start, b):
        pltpu.async_copy(
            out_v[b], out_hbm.at[pl.ds(start * EMBED_DIM, _TC * EMBED_DIM)],
            sem_o[b],
        )

    def out_wait(start, b):
        pltpu.make_async_copy(
            out_v[b], out_hbm.at[pl.ds(start * EMBED_DIM, _TC * EMBED_DIM)],
            sem_o[b],
        ).wait()

    def transpose_body(b):
        @pl.loop(0, EMBED_DIM)
        def _(d):
            for t16 in range(_TC // 16):
                v = in_v[b][d, pl.ds(t16 * 16, 16)]
                idx = iota64 + (t16 * 16 * EMBED_DIM + d)
                plsc.store_scatter(out_v[b], [idx], v)

    def chunk_start(c):
        return c * _TC

    in_start(chunk_start(wid), 0)
    in_start(chunk_start(_NW + wid), 1)

    @pl.loop(0, _KPW - 2, step=2)
    def _(k):
        for b in range(2):
            c = (k + b) * _NW + wid
            start = chunk_start(c)
            in_wait(start, b)

            @pl.when(k + b >= 2)
            def _():
                out_wait(chunk_start((k + b - 2) * _NW + wid), b)

            transpose_body(b)
            out_start(start, b)
            in_start(chunk_start((k + b + 2) * _NW + wid), b)

    for b in range(2):
        c = (_KPW - 2 + b) * _NW + wid
        start = chunk_start(c)
        in_wait(start, b)
        out_wait(chunk_start((_KPW - 4 + b) * _NW + wid), b)
        transpose_body(b)
        out_start(start, b)

    @pl.when(wid < _NEXTRA)
    def _():
        start = chunk_start(_KPW * _NW + wid)
        in_start(start, 0)
        in_wait(start, 0)
        out_wait(chunk_start((_KPW - 2) * _NW + wid), 0)
        transpose_body(0)
        out_start(start, 0)
        out_wait(start, 0)

    @pl.when(wid >= _NEXTRA)
    def _():
        out_wait(chunk_start((_KPW - 2) * _NW + wid), 0)

    out_wait(chunk_start((_KPW - 1) * _NW + wid), 1)

    @pl.when(wid == _NEXTRA)
    def _():
        start = _NFULL * _TC
        pltpu.async_copy(wt_hbm.at[:, pl.ds(start, _TAIL)], in_t, sem_t)
        pltpu.make_async_copy(
            wt_hbm.at[:, pl.ds(start, _TAIL)], in_t, sem_t
        ).wait()

        @pl.loop(0, EMBED_DIM)
        def _(d):
            for t16 in range(_TAIL // 16):
                v = in_t[d, pl.ds(t16 * 16, 16)]
                idx = iota64 + (t16 * 16 * EMBED_DIM + d)
                plsc.store_scatter(out_t, [idx], v)

        pltpu.async_copy(
            out_t, out_hbm.at[pl.ds(start * EMBED_DIM, _TAIL * EMBED_DIM)], sem_t
        )
        pltpu.make_async_copy(
            out_t, out_hbm.at[pl.ds(start * EMBED_DIM, _TAIL * EMBED_DIM)], sem_t
        ).wait()


_N = B * L
_PER_W = _N // _NW
_CHUNK = 256
_NBUF = 4
_NCHUNK = _PER_W // _CHUNK
assert _NCHUNK % _NBUF == 0


def _gather_kernel(idx_hbm, table_hbm, out_hbm, idx_v, rows_v, sems_g, sems_o):
    wid = lax.axis_index("s") * _NC + lax.axis_index("c")
    base = wid * _PER_W

    pltpu.sync_copy(idx_hbm.at[pl.ds(base, _PER_W)], idx_v)

    def gather_start(c, b):
        pltpu.async_copy(
            table_hbm.at[idx_v.at[pl.ds(c * _CHUNK, _CHUNK)]],
            rows_v.at[b],
            sems_g.at[b],
        )

    def gather_wait(c, b):
        pltpu.make_async_copy(
            table_hbm.at[idx_v.at[pl.ds(c * _CHUNK, _CHUNK)]],
            rows_v.at[b],
            sems_g.at[b],
        ).wait()

    def out_start(c, b):
        pltpu.async_copy(
            rows_v.at[b],
            out_hbm.at[pl.ds(base + c * _CHUNK, _CHUNK), pl.ds(0, EMBED_DIM)],
            sems_o.at[b],
        )

    def out_wait(c, b):
        pltpu.make_async_copy(
            rows_v.at[b],
            out_hbm.at[pl.ds(base + c * _CHUNK, _CHUNK), pl.ds(0, EMBED_DIM)],
            sems_o.at[b],
        ).wait()

    for b in range(_NBUF):
        gather_start(b, b)

    @pl.loop(0, _NCHUNK - _NBUF, step=_NBUF)
    def _(g):
        for b in range(_NBUF):
            gather_wait(g + b, b)
            out_start(g + b, b)
        for b in range(_NBUF):
            out_wait(g + b, b)
            gather_start(g + _NBUF + b, b)

    last = _NCHUNK - _NBUF
    for b in range(_NBUF):
        gather_wait(last + b, b)
        out_start(last + b, b)
    for b in range(_NBUF):
        out_wait(last + b, b)


@jax.jit
def kernel(tokens, word_embed_weight):
    idx = tokens.reshape(_N).astype(jnp.int32)
    mesh = plsc.VectorSubcoreMesh(core_axis_name="c", subcore_axis_name="s")

    wt = word_embed_weight.T
    tlin = pl.kernel(
        _transpose_kernel,
        out_type=jax.ShapeDtypeStruct((VOCAB * EMBED_DIM,), jnp.float32),
        mesh=mesh,
        scratch_types=[
            pltpu.VMEM((EMBED_DIM, _TC), jnp.float32),
            pltpu.VMEM((EMBED_DIM, _TC), jnp.float32),
            pltpu.VMEM((_TC * EMBED_DIM,), jnp.float32),
            pltpu.VMEM((_TC * EMBED_DIM,), jnp.float32),
            pltpu.VMEM((EMBED_DIM, _TAIL), jnp.float32),
            pltpu.VMEM((_TAIL * EMBED_DIM,), jnp.float32),
            pltpu.SemaphoreType.DMA,
            pltpu.SemaphoreType.DMA,
            pltpu.SemaphoreType.DMA,
            pltpu.SemaphoreType.DMA,
            pltpu.SemaphoreType.DMA,
        ],
        compiler_params=pltpu.CompilerParams(
            use_tc_tiling_on_sc=True, needs_layout_passes=False
        ),
    )(wt)
    table = tlin.reshape(VOCAB, EMBED_DIM)

    out = pl.kernel(
        _gather_kernel,
        out_type=jax.ShapeDtypeStruct((_N, PAD_DIM), jnp.float32),
        mesh=mesh,
        scratch_types=[
            pltpu.VMEM((_PER_W,), jnp.int32),
            pltpu.VMEM((_NBUF, _CHUNK, EMBED_DIM), jnp.float32),
            pltpu.SemaphoreType.DMA((_NBUF,)),
            pltpu.SemaphoreType.DMA((_NBUF,)),
        ],
        compiler_params=pltpu.CompilerParams(use_tc_tiling_on_sc=False),
    )(idx, table)
    return out[:, :EMBED_DIM].reshape(B, L, EMBED_DIM)


# speedup vs baseline: 1.7090x; 1.7090x over previous
import jax
import jax.numpy as jnp
from jax import lax
from jax.experimental import pallas as pl
from jax.experimental.pallas import tpu as pltpu
from jax.experimental.pallas import tpu_sc as plsc

B = 4096
L = 200
VOCAB = 1000000
EMBED_DIM = 64
PAD_DIM = 128

_info = plsc.get_sparse_core_info()
_NC = _info.num_cores
_NS = _info.num_subcores
_NW = _NC * _NS

_TC = 256
_NFULL = VOCAB // _TC
_KPW = _NFULL // _NW
_NEXTRA = _NFULL - _KPW * _NW
_TAIL = VOCAB - _NFULL * _TC


def _transpose_kernel(wt_hbm, out_hbm, in_v0, in_v1, out_v0, out_v1, in_t, out_t,
                      sem_i0, sem_i1, sem_o0, sem_o1, sem_t):
    in_v = (in_v0, in_v1)
    out_v = (out_v0, out_v1)
    sem_i = (sem_i0, sem_i1)
    sem_o = (sem_o0, sem_o1)
    wid = lax.axis_index("s") * _NC + lax.axis_index("c")

    iota16 = lax.iota(jnp.int32, 16)
    rot = [(iota16 + j) % 16 for j in range(16)]
    rot_out = [iota16 * EMBED_DIM + rot[j] for j in range(16)]

    def in_start(start, b):
        pltpu.async_copy(wt_hbm.at[:, pl.ds(start, _TC)], in_v[b], sem_i[b])

    def in_wait(start, b):
        pltpu.make_async_copy(
            wt_hbm.at[:, pl.ds(start, _TC)], in_v[b], sem_i[b]
        ).wait()

    def out_start(start, b):
        pltpu.async_copy(
            out_v[b], out_hbm.at[pl.ds(start * EMBED_DIM, _TC * EMBED_DIM)],
            sem_o[b],
        )

    def out_wait(start, b):
        pltpu.make_async_copy(
            out_v[b], out_hbm.at[pl.ds(start * EMBED_DIM, _TC * EMBED_DIM)],
            sem_o[b],
        ).wait()

    def transpose_body(b):
        @pl.loop(0, _TC // 16)
        def _(tb):
            t0 = tb * 16
            col_idx = iota16 + t0
            for d0 in range(0, EMBED_DIM, 16):
                for j in range(16):
                    v = plsc.load_gather(in_v[b], [rot[j] + d0, col_idx])
                    plsc.store_scatter(
                        out_v[b], [rot_out[j] + (t0 * EMBED_DIM + d0)], v
                    )

    def chunk_start(c):
        return c * _TC

    in_start(chunk_start(wid), 0)
    in_start(chunk_start(_NW + wid), 1)

    @pl.loop(0, _KPW - 2, step=2)
    def _(k):
        for b in range(2):
            c = (k + b) * _NW + wid
            start = chunk_start(c)
            in_wait(start, b)

            @pl.when(k + b >= 2)
            def _():
                out_wait(chunk_start((k + b - 2) * _NW + wid), b)

            transpose_body(b)
            out_start(start, b)
            in_start(chunk_start((k + b + 2) * _NW + wid), b)

    for b in range(2):
        c = (_KPW - 2 + b) * _NW + wid
        start = chunk_start(c)
        in_wait(start, b)
        out_wait(chunk_start((_KPW - 4 + b) * _NW + wid), b)
        transpose_body(b)
        out_start(start, b)

    @pl.when(wid < _NEXTRA)
    def _():
        start = chunk_start(_KPW * _NW + wid)
        in_start(start, 0)
        in_wait(start, 0)
        out_wait(chunk_start((_KPW - 2) * _NW + wid), 0)
        transpose_body(0)
        out_start(start, 0)
        out_wait(start, 0)

    @pl.when(wid >= _NEXTRA)
    def _():
        out_wait(chunk_start((_KPW - 2) * _NW + wid), 0)

    out_wait(chunk_start((_KPW - 1) * _NW + wid), 1)

    @pl.when(wid == _NEXTRA)
    def _():
        start = _NFULL * _TC
        pltpu.async_copy(wt_hbm.at[:, pl.ds(start, _TAIL)], in_t, sem_t)
        pltpu.make_async_copy(
            wt_hbm.at[:, pl.ds(start, _TAIL)], in_t, sem_t
        ).wait()

        @pl.loop(0, _TAIL // 16)
        def _(tb):
            t0 = tb * 16
            col_idx = iota16 + t0
            for d0 in range(0, EMBED_DIM, 16):
                for j in range(16):
                    v = plsc.load_gather(in_t, [rot[j] + d0, col_idx])
                    plsc.store_scatter(
                        out_t, [rot_out[j] + (t0 * EMBED_DIM + d0)], v
                    )

        pltpu.async_copy(
            out_t, out_hbm.at[pl.ds(start * EMBED_DIM, _TAIL * EMBED_DIM)], sem_t
        )
        pltpu.make_async_copy(
            out_t, out_hbm.at[pl.ds(start * EMBED_DIM, _TAIL * EMBED_DIM)], sem_t
        ).wait()


_N = B * L
_PER_W = _N // _NW
_CHUNK = 256
_NBUF = 4
_NCHUNK = _PER_W // _CHUNK
assert _NCHUNK % _NBUF == 0


def _gather_kernel(idx_hbm, table_hbm, out_hbm, idx_v, rows_v, sems_g, sems_o):
    wid = lax.axis_index("s") * _NC + lax.axis_index("c")
    base = wid * _PER_W

    pltpu.sync_copy(idx_hbm.at[pl.ds(base, _PER_W)], idx_v)

    def gather_start(c, b):
        pltpu.async_copy(
            table_hbm.at[idx_v.at[pl.ds(c * _CHUNK, _CHUNK)]],
            rows_v.at[b],
            sems_g.at[b],
        )

    def gather_wait(c, b):
        pltpu.make_async_copy(
            table_hbm.at[idx_v.at[pl.ds(c * _CHUNK, _CHUNK)]],
            rows_v.at[b],
            sems_g.at[b],
        ).wait()

    def out_start(c, b):
        pltpu.async_copy(
            rows_v.at[b],
            out_hbm.at[pl.ds(base + c * _CHUNK, _CHUNK), pl.ds(0, EMBED_DIM)],
            sems_o.at[b],
        )

    def out_wait(c, b):
        pltpu.make_async_copy(
            rows_v.at[b],
            out_hbm.at[pl.ds(base + c * _CHUNK, _CHUNK), pl.ds(0, EMBED_DIM)],
            sems_o.at[b],
        ).wait()

    for b in range(_NBUF):
        gather_start(b, b)

    @pl.loop(0, _NCHUNK - _NBUF, step=_NBUF)
    def _(g):
        for b in range(_NBUF):
            gather_wait(g + b, b)
            out_start(g + b, b)
        for b in range(_NBUF):
            out_wait(g + b, b)
            gather_start(g + _NBUF + b, b)

    last = _NCHUNK - _NBUF
    for b in range(_NBUF):
        gather_wait(last + b, b)
        out_start(last + b, b)
    for b in range(_NBUF):
        out_wait(last + b, b)


@jax.jit
def kernel(tokens, word_embed_weight):
    idx = tokens.reshape(_N).astype(jnp.int32)
    mesh = plsc.VectorSubcoreMesh(core_axis_name="c", subcore_axis_name="s")

    wt = word_embed_weight.T
    tlin = pl.kernel(
        _transpose_kernel,
        out_type=jax.ShapeDtypeStruct((VOCAB * EMBED_DIM,), jnp.float32),
        mesh=mesh,
        scratch_types=[
            pltpu.VMEM((EMBED_DIM, _TC), jnp.float32),
            pltpu.VMEM((EMBED_DIM, _TC), jnp.float32),
            pltpu.VMEM((_TC * EMBED_DIM,), jnp.float32),
            pltpu.VMEM((_TC * EMBED_DIM,), jnp.float32),
            pltpu.VMEM((EMBED_DIM, _TAIL), jnp.float32),
            pltpu.VMEM((_TAIL * EMBED_DIM,), jnp.float32),
            pltpu.SemaphoreType.DMA,
            pltpu.SemaphoreType.DMA,
            pltpu.SemaphoreType.DMA,
            pltpu.SemaphoreType.DMA,
            pltpu.SemaphoreType.DMA,
        ],
        compiler_params=pltpu.CompilerParams(
            use_tc_tiling_on_sc=True, needs_layout_passes=False
        ),
    )(wt)
    table = tlin.reshape(VOCAB, EMBED_DIM)

    out = pl.kernel(
        _gather_kernel,
        out_type=jax.ShapeDtypeStruct((_N, PAD_DIM), jnp.float32),
        mesh=mesh,
        scratch_types=[
            pltpu.VMEM((_PER_W,), jnp.int32),
            pltpu.VMEM((_NBUF, _CHUNK, EMBED_DIM), jnp.float32),
            pltpu.SemaphoreType.DMA((_NBUF,)),
            pltpu.SemaphoreType.DMA((_NBUF,)),
        ],
        compiler_params=pltpu.CompilerParams(use_tc_tiling_on_sc=False),
    )(idx, table)
    return out[:, :EMBED_DIM].reshape(B, L, EMBED_DIM)


# speedup vs baseline: 1.8385x; 1.0758x over previous
import jax
import jax.numpy as jnp
from jax import lax
from jax.experimental import pallas as pl
from jax.experimental.pallas import tpu as pltpu
from jax.experimental.pallas import tpu_sc as plsc

B = 4096
L = 200
VOCAB = 1000000
EMBED_DIM = 64
PAD_DIM = 128

_info = plsc.get_sparse_core_info()
_NC = _info.num_cores
_NS = _info.num_subcores
_NW = _NC * _NS

_TC = 256
_NFULL = VOCAB // _TC
_KPW = _NFULL // _NW
_NEXTRA = _NFULL - _KPW * _NW
_TAIL = VOCAB - _NFULL * _TC


def _transpose_kernel(wt_hbm, out_hbm, in_v0, in_v1, out_v0, out_v1, in_t, out_t,
                      sem_i0, sem_i1, sem_o0, sem_o1, sem_t):
    in_v = (in_v0, in_v1)
    out_v = (out_v0, out_v1)
    sem_i = (sem_i0, sem_i1)
    sem_o = (sem_o0, sem_o1)
    wid = lax.axis_index("s") * _NC + lax.axis_index("c")

    iota16 = lax.iota(jnp.int32, 16)
    iota64 = iota16 * EMBED_DIM

    def in_start(start, b):
        pltpu.async_copy(wt_hbm.at[:, pl.ds(start, _TC)], in_v[b], sem_i[b])

    def in_wait(start, b):
        pltpu.make_async_copy(
            wt_hbm.at[:, pl.ds(start, _TC)], in_v[b], sem_i[b]
        ).wait()

    def out_start(start, b):
        pltpu.async_copy(
            out_v[b], out_hbm.at[pl.ds(start * EMBED_DIM, _TC * EMBED_DIM)],
            sem_o[b],
        )

    def out_wait(start, b):
        pltpu.make_async_copy(
            out_v[b], out_hbm.at[pl.ds(start * EMBED_DIM, _TC * EMBED_DIM)],
            sem_o[b],
        ).wait()

    def transpose_body(b):
        @pl.loop(0, _TC // 16)
        def _(tb):
            t0 = tb * 16
            col_idx = iota16 + t0
            for j in range(16):
                rot_j = (iota16 + j) & 15
                out_j = iota64 + rot_j
                for d0 in range(0, EMBED_DIM, 16):
                    v = plsc.load_gather(
                        in_v[b].at[pl.ds(d0, 16)], [rot_j, col_idx]
                    )
                    plsc.store_scatter(
                        out_v[b], [out_j + (t0 * EMBED_DIM + d0)], v
                    )

    def chunk_start(c):
        return c * _TC

    in_start(chunk_start(wid), 0)
    in_start(chunk_start(_NW + wid), 1)

    @pl.loop(0, _KPW - 2, step=2)
    def _(k):
        for b in range(2):
            c = (k + b) * _NW + wid
            start = chunk_start(c)
            in_wait(start, b)

            @pl.when(k + b >= 2)
            def _():
                out_wait(chunk_start((k + b - 2) * _NW + wid), b)

            transpose_body(b)
            out_start(start, b)
            in_start(chunk_start((k + b + 2) * _NW + wid), b)

    for b in range(2):
        c = (_KPW - 2 + b) * _NW + wid
        start = chunk_start(c)
        in_wait(start, b)
        out_wait(chunk_start((_KPW - 4 + b) * _NW + wid), b)
        transpose_body(b)
        out_start(start, b)

    @pl.when(wid < _NEXTRA)
    def _():
        start = chunk_start(_KPW * _NW + wid)
        in_start(start, 0)
        in_wait(start, 0)
        out_wait(chunk_start((_KPW - 2) * _NW + wid), 0)
        transpose_body(0)
        out_start(start, 0)
        out_wait(start, 0)

    @pl.when(wid >= _NEXTRA)
    def _():
        out_wait(chunk_start((_KPW - 2) * _NW + wid), 0)

    out_wait(chunk_start((_KPW - 1) * _NW + wid), 1)

    @pl.when(wid == _NEXTRA)
    def _():
        start = _NFULL * _TC
        pltpu.async_copy(wt_hbm.at[:, pl.ds(start, _TAIL)], in_t, sem_t)
        pltpu.make_async_copy(
            wt_hbm.at[:, pl.ds(start, _TAIL)], in_t, sem_t
        ).wait()

        @pl.loop(0, _TAIL // 16)
        def _(tb):
            t0 = tb * 16
            col_idx = iota16 + t0
            for j in range(16):
                rot_j = (iota16 + j) & 15
                out_j = iota64 + rot_j
                for d0 in range(0, EMBED_DIM, 16):
                    v = plsc.load_gather(in_t.at[pl.ds(d0, 16)], [rot_j, col_idx])
                    plsc.store_scatter(out_t, [out_j + (t0 * EMBED_DIM + d0)], v)

        pltpu.async_copy(
            out_t, out_hbm.at[pl.ds(start * EMBED_DIM, _TAIL * EMBED_DIM)], sem_t
        )
        pltpu.make_async_copy(
            out_t, out_hbm.at[pl.ds(start * EMBED_DIM, _TAIL * EMBED_DIM)], sem_t
        ).wait()


_N = B * L
_PER_W = _N // _NW
_CHUNK = 256
_NBUF = 4
_NCHUNK = _PER_W // _CHUNK
assert _NCHUNK % _NBUF == 0


def _gather_kernel(idx_hbm, table_hbm, out_hbm, idx_v, rows_v, sems_g, sems_o):
    wid = lax.axis_index("s") * _NC + lax.axis_index("c")
    base = wid * _PER_W

    pltpu.sync_copy(idx_hbm.at[pl.ds(base, _PER_W)], idx_v)

    def gather_start(c, b):
        pltpu.async_copy(
            table_hbm.at[idx_v.at[pl.ds(c * _CHUNK, _CHUNK)]],
            rows_v.at[b],
            sems_g.at[b],
        )

    def gather_wait(c, b):
        pltpu.make_async_copy(
            table_hbm.at[idx_v.at[pl.ds(c * _CHUNK, _CHUNK)]],
            rows_v.at[b],
            sems_g.at[b],
        ).wait()

    def out_start(c, b):
        pltpu.async_copy(
            rows_v.at[b],
            out_hbm.at[pl.ds(base + c * _CHUNK, _CHUNK), pl.ds(0, EMBED_DIM)],
            sems_o.at[b],
        )

    def out_wait(c, b):
        pltpu.make_async_copy(
            rows_v.at[b],
            out_hbm.at[pl.ds(base + c * _CHUNK, _CHUNK), pl.ds(0, EMBED_DIM)],
            sems_o.at[b],
        ).wait()

    for b in range(_NBUF):
        gather_start(b, b)

    @pl.loop(0, _NCHUNK - _NBUF, step=_NBUF)
    def _(g):
        for b in range(_NBUF):
            gather_wait(g + b, b)
            out_start(g + b, b)
        for b in range(_NBUF):
            out_wait(g + b, b)
            gather_start(g + _NBUF + b, b)

    last = _NCHUNK - _NBUF
    for b in range(_NBUF):
        gather_wait(last + b, b)
        out_start(last + b, b)
    for b in range(_NBUF):
        out_wait(last + b, b)


@jax.jit
def kernel(tokens, word_embed_weight):
    idx = tokens.reshape(_N).astype(jnp.int32)
    mesh = plsc.VectorSubcoreMesh(core_axis_name="c", subcore_axis_name="s")

    wt = word_embed_weight.T
    tlin = pl.kernel(
        _transpose_kernel,
        out_type=jax.ShapeDtypeStruct((VOCAB * EMBED_DIM,), jnp.float32),
        mesh=mesh,
        scratch_types=[
            pltpu.VMEM((EMBED_DIM, _TC), jnp.float32),
            pltpu.VMEM((EMBED_DIM, _TC), jnp.float32),
            pltpu.VMEM((_TC * EMBED_DIM,), jnp.float32),
            pltpu.VMEM((_TC * EMBED_DIM,), jnp.float32),
            pltpu.VMEM((EMBED_DIM, _TAIL), jnp.float32),
            pltpu.VMEM((_TAIL * EMBED_DIM,), jnp.float32),
            pltpu.SemaphoreType.DMA,
            pltpu.SemaphoreType.DMA,
            pltpu.SemaphoreType.DMA,
            pltpu.SemaphoreType.DMA,
            pltpu.SemaphoreType.DMA,
        ],
        compiler_params=pltpu.CompilerParams(
            use_tc_tiling_on_sc=True, needs_layout_passes=False
        ),
    )(wt)
    table = tlin.reshape(VOCAB, EMBED_DIM)

    out = pl.kernel(
        _gather_kernel,
        out_type=jax.ShapeDtypeStruct((_N, PAD_DIM), jnp.float32),
        mesh=mesh,
        scratch_types=[
            pltpu.VMEM((_PER_W,), jnp.int32),
            pltpu.VMEM((_NBUF, _CHUNK, EMBED_DIM), jnp.float32),
            pltpu.SemaphoreType.DMA((_NBUF,)),
            pltpu.SemaphoreType.DMA((_NBUF,)),
        ],
        compiler_params=pltpu.CompilerParams(use_tc_tiling_on_sc=False),
    )(idx, table)
    return out[:, :EMBED_DIM].reshape(B, L, EMBED_DIM)


# speedup vs baseline: 2.9000x; 1.5773x over previous
import jax
import jax.numpy as jnp
from jax import lax
from jax.experimental import pallas as pl
from jax.experimental.pallas import tpu as pltpu
from jax.experimental.pallas import tpu_sc as plsc

B = 4096
L = 200
VOCAB = 1000000
EMBED_DIM = 64
PAD_DIM = 128

_info = plsc.get_sparse_core_info()
_NC = _info.num_cores
_NS = _info.num_subcores
_NW = _NC * _NS

_TC = 256
_NFULL = VOCAB // _TC
_KPW = _NFULL // _NW
_NEXTRA = _NFULL - _KPW * _NW
_TAIL = VOCAB - _NFULL * _TC


def _transpose_kernel(wt_hbm, out_hbm, in_v0, in_v1, out_v0, out_v1, in_t, out_t,
                      sem_i0, sem_i1, sem_o0, sem_o1, sem_t):
    in_v = (in_v0, in_v1)
    out_v = (out_v0, out_v1)
    sem_i = (sem_i0, sem_i1)
    sem_o = (sem_o0, sem_o1)
    wid = lax.axis_index("s") * _NC + lax.axis_index("c")

    iota16 = lax.iota(jnp.int32, 16)
    iota64 = iota16 * EMBED_DIM

    def in_start(start, b):
        pltpu.async_copy(wt_hbm.at[:, pl.ds(start, _TC)], in_v[b], sem_i[b])

    def in_wait(start, b):
        pltpu.make_async_copy(
            wt_hbm.at[:, pl.ds(start, _TC)], in_v[b], sem_i[b]
        ).wait()

    def out_start(start, b):
        pltpu.async_copy(
            out_v[b], out_hbm.at[pl.ds(start * EMBED_DIM, _TC * EMBED_DIM)],
            sem_o[b],
        )

    def out_wait(start, b):
        pltpu.make_async_copy(
            out_v[b], out_hbm.at[pl.ds(start * EMBED_DIM, _TC * EMBED_DIM)],
            sem_o[b],
        ).wait()

    def transpose_body(b):
        @pl.loop(0, _TC // 16)
        def _(tb):
            t0 = tb * 16
            col_idx = iota16 + t0
            for j2 in range(0, 16, 2):
                batch = []
                for j in (j2, j2 + 1):
                    rot_j = (iota16 + j) & 15
                    out_j = iota64 + rot_j
                    for d0 in range(0, EMBED_DIM, 16):
                        v = plsc.load_gather(
                            in_v[b].at[pl.ds(d0, 16)], [rot_j, col_idx]
                        )
                        batch.append((out_j + (t0 * EMBED_DIM + d0), v))
                for oidx, v in batch:
                    plsc.store_scatter(out_v[b], [oidx], v)

    def chunk_start(c):
        return c * _TC

    in_start(chunk_start(wid), 0)
    in_start(chunk_start(_NW + wid), 1)

    @pl.loop(0, _KPW - 2, step=2)
    def _(k):
        for b in range(2):
            c = (k + b) * _NW + wid
            start = chunk_start(c)
            in_wait(start, b)

            @pl.when(k + b >= 2)
            def _():
                out_wait(chunk_start((k + b - 2) * _NW + wid), b)

            transpose_body(b)
            out_start(start, b)
            in_start(chunk_start((k + b + 2) * _NW + wid), b)

    for b in range(2):
        c = (_KPW - 2 + b) * _NW + wid
        start = chunk_start(c)
        in_wait(start, b)
        out_wait(chunk_start((_KPW - 4 + b) * _NW + wid), b)
        transpose_body(b)
        out_start(start, b)

    @pl.when(wid < _NEXTRA)
    def _():
        start = chunk_start(_KPW * _NW + wid)
        in_start(start, 0)
        in_wait(start, 0)
        out_wait(chunk_start((_KPW - 2) * _NW + wid), 0)
        transpose_body(0)
        out_start(start, 0)
        out_wait(start, 0)

    @pl.when(wid >= _NEXTRA)
    def _():
        out_wait(chunk_start((_KPW - 2) * _NW + wid), 0)

    out_wait(chunk_start((_KPW - 1) * _NW + wid), 1)

    @pl.when(wid == _NEXTRA)
    def _():
        start = _NFULL * _TC
        pltpu.async_copy(wt_hbm.at[:, pl.ds(start, _TAIL)], in_t, sem_t)
        pltpu.make_async_copy(
            wt_hbm.at[:, pl.ds(start, _TAIL)], in_t, sem_t
        ).wait()

        @pl.loop(0, _TAIL // 16)
        def _(tb):
            t0 = tb * 16
            col_idx = iota16 + t0
            for j2 in range(0, 16, 2):
                batch = []
                for j in (j2, j2 + 1):
                    rot_j = (iota16 + j) & 15
                    out_j = iota64 + rot_j
                    for d0 in range(0, EMBED_DIM, 16):
                        v = plsc.load_gather(
                            in_t.at[pl.ds(d0, 16)], [rot_j, col_idx]
                        )
                        batch.append((out_j + (t0 * EMBED_DIM + d0), v))
                for oidx, v in batch:
                    plsc.store_scatter(out_t, [oidx], v)

        pltpu.async_copy(
            out_t, out_hbm.at[pl.ds(start * EMBED_DIM, _TAIL * EMBED_DIM)], sem_t
        )
        pltpu.make_async_copy(
            out_t, out_hbm.at[pl.ds(start * EMBED_DIM, _TAIL * EMBED_DIM)], sem_t
        ).wait()


_N = B * L
_PER_W = _N // _NW
_CHUNK = 256
_NBUF = 4
_NCHUNK = _PER_W // _CHUNK
assert _NCHUNK % _NBUF == 0


def _gather_kernel(idx_hbm, table_hbm, out_hbm, idx_v, rows_v, sems_g, sems_o):
    wid = lax.axis_index("s") * _NC + lax.axis_index("c")
    base = wid * _PER_W

    pltpu.sync_copy(idx_hbm.at[pl.ds(base, _PER_W)], idx_v)

    def gather_start(c, b):
        pltpu.async_copy(
            table_hbm.at[idx_v.at[pl.ds(c * _CHUNK, _CHUNK)]],
            rows_v.at[b],
            sems_g.at[b],
        )

    def gather_wait(c, b):
        pltpu.make_async_copy(
            table_hbm.at[idx_v.at[pl.ds(c * _CHUNK, _CHUNK)]],
            rows_v.at[b],
            sems_g.at[b],
        ).wait()

    def out_start(c, b):
        pltpu.async_copy(
            rows_v.at[b],
            out_hbm.at[pl.ds(base + c * _CHUNK, _CHUNK), pl.ds(0, EMBED_DIM)],
            sems_o.at[b],
        )

    def out_wait(c, b):
        pltpu.make_async_copy(
            rows_v.at[b],
            out_hbm.at[pl.ds(base + c * _CHUNK, _CHUNK), pl.ds(0, EMBED_DIM)],
            sems_o.at[b],
        ).wait()

    for b in range(_NBUF):
        gather_start(b, b)

    @pl.loop(0, _NCHUNK - _NBUF, step=_NBUF)
    def _(g):
        for b in range(_NBUF):
            gather_wait(g + b, b)
            out_start(g + b, b)
        for b in range(_NBUF):
            out_wait(g + b, b)
            gather_start(g + _NBUF + b, b)

    last = _NCHUNK - _NBUF
    for b in range(_NBUF):
        gather_wait(last + b, b)
        out_start(last + b, b)
    for b in range(_NBUF):
        out_wait(last + b, b)


@jax.jit
def kernel(tokens, word_embed_weight):
    idx = tokens.reshape(_N).astype(jnp.int32)
    mesh = plsc.VectorSubcoreMesh(core_axis_name="c", subcore_axis_name="s")

    wt = word_embed_weight.T
    tlin = pl.kernel(
        _transpose_kernel,
        out_type=jax.ShapeDtypeStruct((VOCAB * EMBED_DIM,), jnp.float32),
        mesh=mesh,
        scratch_types=[
            pltpu.VMEM((EMBED_DIM, _TC), jnp.float32),
            pltpu.VMEM((EMBED_DIM, _TC), jnp.float32),
            pltpu.VMEM((_TC * EMBED_DIM,), jnp.float32),
            pltpu.VMEM((_TC * EMBED_DIM,), jnp.float32),
            pltpu.VMEM((EMBED_DIM, _TAIL), jnp.float32),
            pltpu.VMEM((_TAIL * EMBED_DIM,), jnp.float32),
            pltpu.SemaphoreType.DMA,
            pltpu.SemaphoreType.DMA,
            pltpu.SemaphoreType.DMA,
            pltpu.SemaphoreType.DMA,
            pltpu.SemaphoreType.DMA,
        ],
        compiler_params=pltpu.CompilerParams(
            use_tc_tiling_on_sc=True, needs_layout_passes=False
        ),
    )(wt)
    table = tlin.reshape(VOCAB, EMBED_DIM)

    out = pl.kernel(
        _gather_kernel,
        out_type=jax.ShapeDtypeStruct((_N, PAD_DIM), jnp.float32),
        mesh=mesh,
        scratch_types=[
            pltpu.VMEM((_PER_W,), jnp.int32),
            pltpu.VMEM((_NBUF, _CHUNK, EMBED_DIM), jnp.float32),
            pltpu.SemaphoreType.DMA((_NBUF,)),
            pltpu.SemaphoreType.DMA((_NBUF,)),
        ],
        compiler_params=pltpu.CompilerParams(use_tc_tiling_on_sc=False),
    )(idx, table)
    return out[:, :EMBED_DIM].reshape(B, L, EMBED_DIM)
